# baseline (speedup 1.0000x reference)
import functools

import jax
import jax.numpy as jnp
from jax import lax
from jax.experimental import pallas as pl
from jax.experimental.pallas import tpu as pltpu
from jax.experimental.pallas import tpu_sc as plsc

_B = 4096
_NC = 2
_NS = 16
_NW = _NC * _NS
_BPW = _B // _NW
_L = 16


def _sc_body(user_hbm, item_hbm, uemb_hbm, iemb_hbm, ubias_hbm, ibias_hbm,
             a_out, d_out, a_loc, d_loc):
    wid = lax.axis_index("s") * _NC + lax.axis_index("c")
    base = wid * _BPW
    for c in range(_BPW // _L):
        sl = pl.ds(c * _L, _L)
        a_loc[sl] = jnp.full((_L,), 1.0, jnp.float32)
        d_loc[sl] = jnp.full((_L,), 2.0, jnp.float32)
    pltpu.sync_copy(a_loc, a_out.at[pl.ds(base, _BPW)])
    pltpu.sync_copy(d_loc, d_out.at[pl.ds(base, _BPW)])


_sc_gather = pl.kernel(
    _sc_body,
    out_type=(jax.ShapeDtypeStruct((_B,), jnp.float32),
              jax.ShapeDtypeStruct((_B,), jnp.float32)),
    mesh=plsc.VectorSubcoreMesh(core_axis_name="c", subcore_axis_name="s"),
    compiler_params=pltpu.CompilerParams(needs_layout_passes=False),
    scratch_types=[
        pltpu.VMEM((_BPW,), jnp.float32),
        pltpu.VMEM((_BPW,), jnp.float32),
    ],
)


def kernel(user, item, user_embeddings, item_embeddings, user_biases, item_biases):
    user = user.astype(jnp.int32)
    item = item.astype(jnp.int32)
    a, d = _sc_gather(user, item, user_embeddings, item_embeddings,
                      user_biases, item_biases)
    return (a, d)


# speedup vs baseline: 1.4199x; 1.4199x over previous
import functools

import jax
import jax.numpy as jnp
from jax import lax
from jax.experimental import pallas as pl
from jax.experimental.pallas import tpu as pltpu
from jax.experimental.pallas import tpu_sc as plsc

_B = 4096
_D = 32
_MEAN = 3.5
_NC = 2
_NS = 16
_NW = _NC * _NS
_BPW = _B // _NW
_L = 16
_NV = 1000000
_BROWS = (_NV + 127) // 128
_BPAD = _BROWS * 128 - _NV


def _sc_body(user_hbm, item_hbm, uemb_hbm, iemb_hbm, ubias_hbm, ibias_hbm,
             a_out, d_out,
             uidx_v, iidx_v, ubrow_v, ibrow_v,
             ur_v, ir_v, ubr_v, ibr_v, a_loc, d_loc, sem):
    wid = lax.axis_index("s") * _NC + lax.axis_index("c")
    base = wid * _BPW

    pltpu.sync_copy(user_hbm.at[pl.ds(base, _BPW)], uidx_v)
    pltpu.sync_copy(item_hbm.at[pl.ds(base, _BPW)], iidx_v)
    for c in range(_BPW // _L):
        sl = pl.ds(c * _L, _L)
        ubrow_v[sl] = lax.shift_right_logical(uidx_v[sl], 7)
        ibrow_v[sl] = lax.shift_right_logical(iidx_v[sl], 7)

    copies = [
        pltpu.async_copy(ubias_hbm.at[ubrow_v], ubr_v, sem),
        pltpu.async_copy(ibias_hbm.at[ibrow_v], ibr_v, sem),
    ]
    for c in range(_BPW // _L):
        sl = pl.ds(c * _L, _L)
        u16 = uidx_v[sl]
        i16 = iidx_v[sl]
        for j in range(_L):
            r = c * _L + j
            copies.append(pltpu.async_copy(
                uemb_hbm.at[pl.ds(u16[j], 1), :], ur_v.at[pl.ds(r, 1), :], sem))
            copies.append(pltpu.async_copy(
                iemb_hbm.at[pl.ds(i16[j], 1), :], ir_v.at[pl.ds(r, 1), :], sem))
    for cp in copies:
        cp.wait()

    lane = lax.iota(jnp.int32, _L)
    for g in range(_BPW // _L):
        sl = pl.ds(g * _L, _L)
        row = g * _L + lane
        acc = jnp.zeros((_L,), jnp.float32)
        for k in range(_D):
            col = jnp.full((_L,), k, jnp.int32)
            acc = acc + (plsc.load_gather(ur_v, [row, col])
                         * plsc.load_gather(ir_v, [row, col]))
        d_loc[sl] = acc
        u = uidx_v[sl]
        i = iidx_v[sl]
        ub = plsc.load_gather(ubr_v, [row, jnp.bitwise_and(u, 127)])
        ib = plsc.load_gather(ibr_v, [row, jnp.bitwise_and(i, 127)])
        a_loc[sl] = ub + ib

    pltpu.sync_copy(a_loc, a_out.at[pl.ds(base, _BPW)])
    pltpu.sync_copy(d_loc, d_out.at[pl.ds(base, _BPW)])


_sc_gather = pl.kernel(
    _sc_body,
    out_type=(jax.ShapeDtypeStruct((_B,), jnp.float32),
              jax.ShapeDtypeStruct((_B,), jnp.float32)),
    mesh=plsc.VectorSubcoreMesh(core_axis_name="c", subcore_axis_name="s"),
    compiler_params=pltpu.CompilerParams(needs_layout_passes=False,
                                         use_tc_tiling_on_sc=True),
    scratch_types=[
        pltpu.VMEM((_BPW,), jnp.int32),
        pltpu.VMEM((_BPW,), jnp.int32),
        pltpu.VMEM((_BPW,), jnp.int32),
        pltpu.VMEM((_BPW,), jnp.int32),
        pltpu.VMEM((_BPW, _D), jnp.float32),
        pltpu.VMEM((_BPW, _D), jnp.float32),
        pltpu.VMEM((_BPW, 128), jnp.float32),
        pltpu.VMEM((_BPW, 128), jnp.float32),
        pltpu.VMEM((_BPW,), jnp.float32),
        pltpu.VMEM((_BPW,), jnp.float32),
        pltpu.SemaphoreType.DMA,
    ],
)

_ROWS = 512


def _bcast_body(a_ref, d_ref, o_ref):
    o_ref[...] = a_ref[...] + d_ref[...] + _MEAN


_bcast = pl.pallas_call(
    _bcast_body,
    grid=(_B // _ROWS,),
    in_specs=[
        pl.BlockSpec((_ROWS, 1), lambda i: (i, 0)),
        pl.BlockSpec((1, _B), lambda i: (0, 0)),
    ],
    out_specs=pl.BlockSpec((_ROWS, _B), lambda i: (i, 0)),
    out_shape=jax.ShapeDtypeStruct((_B, _B), jnp.float32),
)


def kernel(user, item, user_embeddings, item_embeddings, user_biases, item_biases):
    user = user.astype(jnp.int32)
    item = item.astype(jnp.int32)
    ub1 = jnp.pad(user_biases.reshape(-1), (0, _BPAD)).reshape(_BROWS, 128)
    ib1 = jnp.pad(item_biases.reshape(-1), (0, _BPAD)).reshape(_BROWS, 128)
    a, d = _sc_gather(user, item, user_embeddings, item_embeddings, ub1, ib1)
    return _bcast(a.reshape(_B, 1), d.reshape(1, _B))


# speedup vs baseline: 5.2769x; 3.7165x over previous
import functools

import jax
import jax.numpy as jnp
from jax import lax
from jax.experimental import pallas as pl
from jax.experimental.pallas import tpu as pltpu
from jax.experimental.pallas import tpu_sc as plsc

_B = 4096
_D = 32
_MEAN = 3.5
_NC = 2
_NS = 16
_NW = _NC * _NS
_BPW = _B // _NW
_L = 16
_NV = 1000000
_BROWS = (_NV + 127) // 128
_BPAD = _BROWS * 128 - _NV
_CH = 4
_NCH = _BPW // _CH


def _sc_body(user_hbm, item_hbm, uembT_hbm, iembT_hbm, ubias_hbm, ibias_hbm,
             a_out, d_out,
             uidx_v, iidx_v, ubrow_v, ibrow_v,
             bu_v, bi_v, ubr_v, ibr_v, tmp_v, a_loc, d_loc,
             sem0, sem1, semb):
    wid = lax.axis_index("s") * _NC + lax.axis_index("c")
    base = wid * _BPW

    pltpu.sync_copy(user_hbm.at[pl.ds(base, _BPW)], uidx_v)
    pltpu.sync_copy(item_hbm.at[pl.ds(base, _BPW)], iidx_v)

    for c in range(_BPW // _L):
        sl = pl.ds(c * _L, _L)
        ubrow_v[sl] = lax.shift_right_logical(uidx_v[sl], 7)
        ibrow_v[sl] = lax.shift_right_logical(iidx_v[sl], 7)
    bias_cps = [
        pltpu.async_copy(ubias_hbm.at[ubrow_v], ubr_v, semb),
        pltpu.async_copy(ibias_hbm.at[ibrow_v], ibr_v, semb),
    ]

    uscal, iscal = [], []
    for g in range(_BPW // _L):
        u16 = uidx_v[pl.ds(g * _L, _L)]
        i16 = iidx_v[pl.ds(g * _L, _L)]
        for j in range(_L):
            uscal.append(u16[j])
            iscal.append(i16[j])

    sems = (sem0, sem1)

    def fire(c):
        ring = c % 2
        sem = sems[ring]
        cps = []
        for s in range(_CH):
            e = c * _CH + s
            uoff = pl.multiple_of(jnp.bitwise_and(uscal[e], -128), 128)
            ioff = pl.multiple_of(jnp.bitwise_and(iscal[e], -128), 128)
            cps.append(pltpu.async_copy(
                uembT_hbm.at[:, pl.ds(uoff, 128)], bu_v.at[ring, s], sem))
            cps.append(pltpu.async_copy(
                iembT_hbm.at[:, pl.ds(ioff, 128)], bi_v.at[ring, s], sem))
        return cps

    lane = lax.iota(jnp.int32, _L)
    s_vec = lax.shift_right_logical(lane, 2)
    kp8 = lax.shift_left(jnp.bitwise_and(lane, 3), 3)
    msk = jnp.bitwise_and(lane, 3) == 0

    pending = fire(0)
    for c in range(_NCH):
        nxt = fire(c + 1) if c + 1 < _NCH else []
        for cp in pending:
            cp.wait()
        pending = nxt

        ring = c % 2
        cb4 = jnp.int32(c * _CH)
        bu16 = jnp.bitwise_and(plsc.load_gather(uidx_v, [cb4 + s_vec]), 127)
        bi16 = jnp.bitwise_and(plsc.load_gather(iidx_v, [cb4 + s_vec]), 127)
        acc = jnp.zeros((_L,), jnp.float32)
        bu_r = bu_v.at[ring]
        bi_r = bi_v.at[ring]
        for t in range(8):
            kv = kp8 + t
            acc = acc + (plsc.load_gather(bu_r, [s_vec, kv, bu16])
                         * plsc.load_gather(bi_r, [s_vec, kv, bi16]))
        tmp_v[...] = acc
        tot = (plsc.load_gather(tmp_v, [lax.shift_left(s_vec, 2)])
               + plsc.load_gather(tmp_v, [lax.shift_left(s_vec, 2) + 1])
               + plsc.load_gather(tmp_v, [lax.shift_left(s_vec, 2) + 2])
               + plsc.load_gather(tmp_v, [lax.shift_left(s_vec, 2) + 3]))
        plsc.store_scatter(d_loc, [cb4 + s_vec], tot, mask=msk)

    for cp in bias_cps:
        cp.wait()
    for g in range(_BPW // _L):
        sl = pl.ds(g * _L, _L)
        row = g * _L + lane
        u = uidx_v[sl]
        i = iidx_v[sl]
        ub = plsc.load_gather(ubr_v, [row, jnp.bitwise_and(u, 127)])
        ib = plsc.load_gather(ibr_v, [row, jnp.bitwise_and(i, 127)])
        a_loc[sl] = ub + ib

    pltpu.sync_copy(a_loc, a_out.at[pl.ds(base, _BPW)])
    pltpu.sync_copy(d_loc, d_out.at[pl.ds(base, _BPW)])


_sc_gather = pl.kernel(
    _sc_body,
    out_type=(jax.ShapeDtypeStruct((_B,), jnp.float32),
              jax.ShapeDtypeStruct((_B,), jnp.float32)),
    mesh=plsc.VectorSubcoreMesh(core_axis_name="c", subcore_axis_name="s"),
    compiler_params=pltpu.CompilerParams(needs_layout_passes=False,
                                         use_tc_tiling_on_sc=True),
    scratch_types=[
        pltpu.VMEM((_BPW,), jnp.int32),
        pltpu.VMEM((_BPW,), jnp.int32),
        pltpu.VMEM((_BPW,), jnp.int32),
        pltpu.VMEM((_BPW,), jnp.int32),
        pltpu.VMEM((2, _CH, _D, 128), jnp.float32),
        pltpu.VMEM((2, _CH, _D, 128), jnp.float32),
        pltpu.VMEM((_BPW, 128), jnp.float32),
        pltpu.VMEM((_BPW, 128), jnp.float32),
        pltpu.VMEM((_L,), jnp.float32),
        pltpu.VMEM((_BPW,), jnp.float32),
        pltpu.VMEM((_BPW,), jnp.float32),
        pltpu.SemaphoreType.DMA,
        pltpu.SemaphoreType.DMA,
        pltpu.SemaphoreType.DMA,
    ],
)

_ROWS = 512


def _bcast_body(a_ref, d_ref, o_ref):
    o_ref[...] = a_ref[...] + d_ref[...] + _MEAN


_bcast = pl.pallas_call(
    _bcast_body,
    grid=(_B // _ROWS,),
    in_specs=[
        pl.BlockSpec((_ROWS, 1), lambda i: (i, 0)),
        pl.BlockSpec((1, _B), lambda i: (0, 0)),
    ],
    out_specs=pl.BlockSpec((_ROWS, _B), lambda i: (i, 0)),
    out_shape=jax.ShapeDtypeStruct((_B, _B), jnp.float32),
)


def kernel(user, item, user_embeddings, item_embeddings, user_biases, item_biases):
    user = user.astype(jnp.int32)
    item = item.astype(jnp.int32)
    ub1 = jnp.pad(user_biases, ((0, _BPAD), (0, 0))).reshape(_BROWS, 128)
    ib1 = jnp.pad(item_biases, ((0, _BPAD), (0, 0))).reshape(_BROWS, 128)
    a, d = _sc_gather(user, item,
                      user_embeddings.T, item_embeddings.T, ub1, ib1)
    return _bcast(a.reshape(_B, 1), d.reshape(1, _B))


# speedup vs baseline: 8.9724x; 1.7003x over previous
import functools

import jax
import jax.numpy as jnp
from jax import lax
from jax.experimental import pallas as pl
from jax.experimental.pallas import tpu as pltpu
from jax.experimental.pallas import tpu_sc as plsc

_B = 4096
_D = 32
_MEAN = 3.5
_NC = 2
_NS = 16
_NW = _NC * _NS
_BPW = _B // _NW
_L = 16
_NV = 1000000
_BROWS = (_NV + 127) // 128
_BPAD = _BROWS * 128 - _NV
_CH = 4
_NCH = _BPW // _CH


def _sc_body(user_hbm, item_hbm, uembT_hbm, iembT_hbm, ubias_hbm, ibias_hbm,
             a_out, d_out,
             uidx_v, iidx_v,
             bu_v, bi_v, ubr_v, ibr_v, tmp_v, a_loc, d_loc,
             sem0, sem1):
    wid = lax.axis_index("s") * _NC + lax.axis_index("c")
    base = wid * _BPW

    pltpu.sync_copy(user_hbm.at[pl.ds(base, _BPW)], uidx_v)
    pltpu.sync_copy(item_hbm.at[pl.ds(base, _BPW)], iidx_v)

    uscal, iscal = [], []
    for g in range(_BPW // _L):
        u16 = uidx_v[pl.ds(g * _L, _L)]
        i16 = iidx_v[pl.ds(g * _L, _L)]
        for j in range(_L):
            uscal.append(u16[j])
            iscal.append(i16[j])

    sems = (sem0, sem1)

    def fire(c):
        ring = c % 2
        sem = sems[ring]
        cps = []
        for s in range(_CH):
            e = c * _CH + s
            uoff = pl.multiple_of(jnp.bitwise_and(uscal[e], -128), 128)
            ioff = pl.multiple_of(jnp.bitwise_and(iscal[e], -128), 128)
            cps.append(pltpu.async_copy(
                uembT_hbm.at[:, pl.ds(uoff, 128)], bu_v.at[ring, s], sem))
            cps.append(pltpu.async_copy(
                iembT_hbm.at[:, pl.ds(ioff, 128)], bi_v.at[ring, s], sem))
            cps.append(pltpu.async_copy(
                ubias_hbm.at[:, pl.ds(uoff, 128)], ubr_v.at[ring, s], sem))
            cps.append(pltpu.async_copy(
                ibias_hbm.at[:, pl.ds(ioff, 128)], ibr_v.at[ring, s], sem))
        return cps

    lane = lax.iota(jnp.int32, _L)
    s_vec = lax.shift_right_logical(lane, 2)
    kp8 = lax.shift_left(jnp.bitwise_and(lane, 3), 3)
    msk = jnp.bitwise_and(lane, 3) == 0

    pending = fire(0)
    for c in range(_NCH):
        nxt = fire(c + 1) if c + 1 < _NCH else []
        for cp in pending:
            cp.wait()
        pending = nxt

        ring = c % 2
        cb4 = jnp.int32(c * _CH)
        bu16 = jnp.bitwise_and(plsc.load_gather(uidx_v, [cb4 + s_vec]), 127)
        bi16 = jnp.bitwise_and(plsc.load_gather(iidx_v, [cb4 + s_vec]), 127)
        acc = jnp.zeros((_L,), jnp.float32)
        bu_r = bu_v.at[ring]
        bi_r = bi_v.at[ring]
        for t in range(8):
            kv = kp8 + t
            acc = acc + (plsc.load_gather(bu_r, [s_vec, kv, bu16])
                         * plsc.load_gather(bi_r, [s_vec, kv, bi16]))
        tmp_v[...] = acc
        tot = (plsc.load_gather(tmp_v, [lax.shift_left(s_vec, 2)])
               + plsc.load_gather(tmp_v, [lax.shift_left(s_vec, 2) + 1])
               + plsc.load_gather(tmp_v, [lax.shift_left(s_vec, 2) + 2])
               + plsc.load_gather(tmp_v, [lax.shift_left(s_vec, 2) + 3]))
        plsc.store_scatter(d_loc, [cb4 + s_vec], tot, mask=msk)

        zero16 = jnp.zeros((_L,), jnp.int32)
        ub = plsc.load_gather(ubr_v.at[ring], [s_vec, zero16, bu16])
        ib = plsc.load_gather(ibr_v.at[ring], [s_vec, zero16, bi16])
        plsc.store_scatter(a_loc, [cb4 + s_vec], ub + ib, mask=msk)

    pltpu.sync_copy(a_loc, a_out.at[pl.ds(base, _BPW)])
    pltpu.sync_copy(d_loc, d_out.at[pl.ds(base, _BPW)])


_sc_gather = pl.kernel(
    _sc_body,
    out_type=(jax.ShapeDtypeStruct((_B,), jnp.float32),
              jax.ShapeDtypeStruct((_B,), jnp.float32)),
    mesh=plsc.VectorSubcoreMesh(core_axis_name="c", subcore_axis_name="s"),
    compiler_params=pltpu.CompilerParams(needs_layout_passes=False,
                                         use_tc_tiling_on_sc=True),
    scratch_types=[
        pltpu.VMEM((_BPW,), jnp.int32),
        pltpu.VMEM((_BPW,), jnp.int32),
        pltpu.VMEM((2, _CH, _D, 128), jnp.float32),
        pltpu.VMEM((2, _CH, _D, 128), jnp.float32),
        pltpu.VMEM((2, _CH, 1, 128), jnp.float32),
        pltpu.VMEM((2, _CH, 1, 128), jnp.float32),
        pltpu.VMEM((_L,), jnp.float32),
        pltpu.VMEM((_BPW,), jnp.float32),
        pltpu.VMEM((_BPW,), jnp.float32),
        pltpu.SemaphoreType.DMA,
        pltpu.SemaphoreType.DMA,
    ],
)

_ROWS = 512


def _bcast_body(a_ref, d_ref, o_ref):
    o_ref[...] = a_ref[...] + d_ref[...] + _MEAN


_bcast = pl.pallas_call(
    _bcast_body,
    grid=(_B // _ROWS,),
    in_specs=[
        pl.BlockSpec((_ROWS, 1), lambda i: (i, 0)),
        pl.BlockSpec((1, _B), lambda i: (0, 0)),
    ],
    out_specs=pl.BlockSpec((_ROWS, _B), lambda i: (i, 0)),
    out_shape=jax.ShapeDtypeStruct((_B, _B), jnp.float32),
)


def kernel(user, item, user_embeddings, item_embeddings, user_biases, item_biases):
    user = user.astype(jnp.int32)
    item = item.astype(jnp.int32)
    a, d = _sc_gather(user, item,
                      user_embeddings.T, item_embeddings.T,
                      user_biases.T, item_biases.T)
    return _bcast(a.reshape(_B, 1), d.reshape(1, _B))


# speedup vs baseline: 9.0029x; 1.0034x over previous
import jax
import jax.numpy as jnp
from jax import lax
from jax.experimental import pallas as pl
from jax.experimental.pallas import tpu as pltpu
from jax.experimental.pallas import tpu_sc as plsc

_B = 4096
_D = 32
_MEAN = 3.5
_NC = 2
_NS = 16
_NW = _NC * _NS
_BPW = _B // _NW
_L = 16
_CH = 4
_NCH = _BPW // _CH


def _sc_body(user_hbm, item_hbm, uembT_hbm, iembT_hbm, ubias_hbm, ibias_hbm,
             a_out, d_out,
             uidx_v, iidx_v,
             bu_v, bi_v, ubr_v, ibr_v, tmp_v, a_loc, d_loc,
             sem0, sem1):
    wid = lax.axis_index("s") * _NC + lax.axis_index("c")
    base = wid * _BPW

    pltpu.sync_copy(user_hbm.at[pl.ds(base, _BPW)], uidx_v)
    pltpu.sync_copy(item_hbm.at[pl.ds(base, _BPW)], iidx_v)

    uscal, iscal = [], []
    for g in range(_BPW // _L):
        u16 = uidx_v[pl.ds(g * _L, _L)]
        i16 = iidx_v[pl.ds(g * _L, _L)]
        for j in range(_L):
            uscal.append(u16[j])
            iscal.append(i16[j])

    sems = (sem0, sem1)

    def fire(c):
        ring = c % 2
        sem = sems[ring]
        cps = []
        for s in range(_CH):
            e = c * _CH + s
            uoff = pl.multiple_of(jnp.bitwise_and(uscal[e], -128), 128)
            ioff = pl.multiple_of(jnp.bitwise_and(iscal[e], -128), 128)
            cps.append(pltpu.async_copy(
                uembT_hbm.at[:, pl.ds(uoff, 128)], bu_v.at[ring, s], sem))
            cps.append(pltpu.async_copy(
                iembT_hbm.at[:, pl.ds(ioff, 128)], bi_v.at[ring, s], sem))
            cps.append(pltpu.async_copy(
                ubias_hbm.at[:, pl.ds(uoff, 128)], ubr_v.at[ring, s], sem))
            cps.append(pltpu.async_copy(
                ibias_hbm.at[:, pl.ds(ioff, 128)], ibr_v.at[ring, s], sem))
        return cps

    lane = lax.iota(jnp.int32, _L)
    s_vec = lax.shift_right_logical(lane, 2)
    kp8 = lax.shift_left(jnp.bitwise_and(lane, 3), 3)
    msk = jnp.bitwise_and(lane, 3) == 0

    pending = fire(0)
    for c in range(_NCH):
        nxt = fire(c + 1) if c + 1 < _NCH else []
        for cp in pending:
            cp.wait()
        pending = nxt

        ring = c % 2
        cb4 = jnp.int32(c * _CH)
        bu16 = jnp.bitwise_and(plsc.load_gather(uidx_v, [cb4 + s_vec]), 127)
        bi16 = jnp.bitwise_and(plsc.load_gather(iidx_v, [cb4 + s_vec]), 127)
        acc = jnp.zeros((_L,), jnp.float32)
        bu_r = bu_v.at[ring]
        bi_r = bi_v.at[ring]
        for t in range(8):
            kv = kp8 + t
            acc = acc + (plsc.load_gather(bu_r, [s_vec, kv, bu16])
                         * plsc.load_gather(bi_r, [s_vec, kv, bi16]))
        tmp_v[...] = acc
        tot = (plsc.load_gather(tmp_v, [lax.shift_left(s_vec, 2)])
               + plsc.load_gather(tmp_v, [lax.shift_left(s_vec, 2) + 1])
               + plsc.load_gather(tmp_v, [lax.shift_left(s_vec, 2) + 2])
               + plsc.load_gather(tmp_v, [lax.shift_left(s_vec, 2) + 3]))
        plsc.store_scatter(d_loc, [cb4 + s_vec], tot, mask=msk)

        zero16 = jnp.zeros((_L,), jnp.int32)
        ub = plsc.load_gather(ubr_v.at[ring], [s_vec, zero16, bu16])
        ib = plsc.load_gather(ibr_v.at[ring], [s_vec, zero16, bi16])
        plsc.store_scatter(a_loc, [cb4 + s_vec], ub + ib, mask=msk)

    pltpu.sync_copy(a_loc, a_out.at[pl.ds(base, _BPW)])
    pltpu.sync_copy(d_loc, d_out.at[pl.ds(base, _BPW)])


_sc_gather = pl.kernel(
    _sc_body,
    out_type=(jax.ShapeDtypeStruct((_B,), jnp.float32),
              jax.ShapeDtypeStruct((_B,), jnp.float32)),
    mesh=plsc.VectorSubcoreMesh(core_axis_name="c", subcore_axis_name="s"),
    compiler_params=pltpu.CompilerParams(needs_layout_passes=False,
                                         use_tc_tiling_on_sc=True),
    scratch_types=[
        pltpu.VMEM((_BPW,), jnp.int32),
        pltpu.VMEM((_BPW,), jnp.int32),
        pltpu.VMEM((2, _CH, _D, 128), jnp.float32),
        pltpu.VMEM((2, _CH, _D, 128), jnp.float32),
        pltpu.VMEM((2, _CH, 1, 128), jnp.float32),
        pltpu.VMEM((2, _CH, 1, 128), jnp.float32),
        pltpu.VMEM((_L,), jnp.float32),
        pltpu.VMEM((_BPW,), jnp.float32),
        pltpu.VMEM((_BPW,), jnp.float32),
        pltpu.SemaphoreType.DMA,
        pltpu.SemaphoreType.DMA,
    ],
)

_ROWS = 512


def _bcast_body(a_ref, d_ref, o_ref):
    o_ref[...] = a_ref[...] + d_ref[...] + _MEAN


_bcast = pl.pallas_call(
    _bcast_body,
    grid=(_B // _ROWS,),
    in_specs=[
        pl.BlockSpec((_ROWS, 1), lambda i: (i, 0)),
        pl.BlockSpec((1, _B), lambda i: (0, 0)),
    ],
    out_specs=pl.BlockSpec((_ROWS, _B), lambda i: (i, 0)),
    out_shape=jax.ShapeDtypeStruct((_B, _B), jnp.float32),
)


def kernel(user, item, user_embeddings, item_embeddings, user_biases, item_biases):
    user = user.astype(jnp.int32)
    item = item.astype(jnp.int32)
    a, d = _sc_gather(user, item,
                      user_embeddings.T, item_embeddings.T,
                      user_biases.T, item_biases.T)
    return _bcast(a.reshape(_B, 1), d.reshape(1, _B))
